# Initial kernel scaffold; baseline (speedup 1.0000x reference)
import jax
import jax.numpy as jnp
from jax.experimental import pallas as pl


def kernel(patches, inds):
    raise NotImplementedError("write your pallas kernel here")


# speedup vs baseline: 84.3171x; 84.3171x over previous
import functools

import jax
import jax.numpy as jnp
from jax import lax
from jax.experimental import pallas as pl
from jax.experimental.pallas import tpu as pltpu
from jax.experimental.pallas import tpu_sc as plsc

_PS = 16
_C = 3
_H = 512
_W = 512
_N = 65536
_WP = _W - _PS + 1
_CHW = _C * _H * _W
_PATCH = _C * _PS * _PS

_NC = 2
_NS = 16
_CHUNK = 16
_PER_TILE = _N // (_NC * _NS)
_SLICE = _CHW // _NS


def _off_body(inds_ref, off_ref):
    i = inds_ref[...]
    y0 = i // _WP
    off_ref[...] = y0 * _W + (i - y0 * _WP)


def _compute_offsets(inds):
    inds2 = inds.reshape(_N // 128, 128)
    out = pl.pallas_call(
        _off_body,
        out_shape=jax.ShapeDtypeStruct((_N // 128, 128), jnp.int32),
    )(inds2)
    return out.reshape(_N)


def _sc_body(pf_hbm, off_hbm, out_hbm, dbuf, ibuf, offs_v, canvas):
    cid = lax.axis_index("c")
    sid = lax.axis_index("s")

    def _zero(i, carry):
        dbuf[pl.ds(i * 16, 16)] = jnp.zeros((16,), jnp.float32)
        return carry

    lax.fori_loop(0, (_CHUNK * _PATCH) // 16, _zero, 0)
    for z in range(_SLICE // (_CHUNK * _PATCH)):
        pltpu.sync_copy(
            dbuf, canvas.at[pl.ds(sid * _SLICE + z * _CHUNK * _PATCH,
                                  _CHUNK * _PATCH)])
    plsc.subcore_barrier()

    tile_base = (cid * _NS + sid) * _PER_TILE

    def _chunk(ck, carry):
        base = tile_base + ck * _CHUNK
        pltpu.sync_copy(off_hbm.at[pl.ds(base, _CHUNK)], offs_v)
        pltpu.sync_copy(pf_hbm.at[pl.ds(base * _PATCH, _CHUNK * _PATCH)],
                        dbuf)

        def _patch(j, c2):
            ob = plsc.load_gather(offs_v, [jnp.full((16,), j, jnp.int32)])
            b = ob + lax.broadcasted_iota(jnp.int32, (16,), 0)
            for r in range(_C * _PS):
                ch, dy = r // _PS, r % _PS
                const = ch * (_H * _W) + dy * _W
                ibuf[pl.ds(j * _PATCH + r * 16, 16)] = b + const
            return c2

        lax.fori_loop(0, _CHUNK, _patch, 0)
        pltpu.sync_copy(dbuf, canvas.at[ibuf], add=True)
        return carry

    lax.fori_loop(0, _PER_TILE // _CHUNK, _chunk, 0)
    plsc.subcore_barrier()

    pltpu.sync_copy(canvas.at[pl.ds(sid * _SLICE, _SLICE)],
                    out_hbm.at[cid, pl.ds(sid * _SLICE, _SLICE)])


_sc_fold = functools.partial(
    pl.kernel,
    out_type=jax.ShapeDtypeStruct((_NC, _CHW), jnp.float32),
    mesh=plsc.VectorSubcoreMesh(core_axis_name="c", subcore_axis_name="s"),
    compiler_params=pltpu.CompilerParams(needs_layout_passes=False),
    scratch_types=[
        pltpu.VMEM((_CHUNK * _PATCH,), jnp.float32),
        pltpu.VMEM((_CHUNK * _PATCH,), jnp.int32),
        pltpu.VMEM((_CHUNK,), jnp.int32),
        pltpu.VMEM_SHARED((_CHW,), jnp.float32),
    ],
)(_sc_body)


def _add_body(p_ref, o_ref):
    o_ref[...] = p_ref[0] + p_ref[1]


def _sum_partials(partials):
    p3 = partials.reshape(_NC, _CHW // 128, 128)
    out = pl.pallas_call(
        _add_body,
        out_shape=jax.ShapeDtypeStruct((_CHW // 128, 128), jnp.float32),
    )(p3)
    return out


def kernel(patches, inds):
    pf = patches.reshape(_N * _PATCH)
    offs = _compute_offsets(inds.astype(jnp.int32))
    partials = _sc_fold(pf, offs)
    vid = _sum_partials(partials)
    return vid.reshape(1, _C, _H, _W)
